# baseline (speedup 1.0000x reference)
import functools

import jax
import jax.numpy as jnp
from jax import lax
from jax.experimental import pallas as pl
from jax.experimental.pallas import tpu as pltpu
from jax.experimental.pallas import tpu_sc as plsc

N = 10000
E = 640000
B = 1
H = 64

NC = 2
NS = 16
NW = NC * NS
EW = E // NW
C = 128
NB = 4
NCH = 160
EPAD = NCH * C
ACC1 = NS * 640
ACCR = N + 16

_mesh = functools.partial(
    plsc.VectorSubcoreMesh, core_axis_name="c", subcore_axis_name="s",
    num_cores=NC, num_subcores=NS)

_F32 = jnp.float32
_I32 = jnp.int32


def _wid():
    return lax.axis_index("s") * NC + lax.axis_index("c")


def _seg1_body(src_hbm, dst_hbm, w_hbm, out_hbm, srcv, dstv,
               v0, v1, v2, v3, zbuf, obuf, acc_sh, s0, s1, s2, s3):
    c = lax.axis_index("c")
    s = lax.axis_index("s")
    w = _wid()
    pltpu.sync_copy(src_hbm.at[w], srcv)
    pltpu.sync_copy(dst_hbm.at[w], dstv)
    z = jnp.zeros((16,), _F32)
    for i in range(40):
        zbuf[pl.ds(i * 16, 16)] = z
    pltpu.sync_copy(zbuf, acc_sh.at[pl.ds(s * 640, 640)])
    plsc.subcore_barrier()

    bufs = (v0, v1, v2, v3)
    sems = (s0, s1, s2, s3)
    for b in range(NB):
        pltpu.async_copy(w_hbm.at[srcv.at[b]], bufs[b], sems[b])

    def chunk(jj, _):
        for b in range(NB):
            j = jj * NB + b
            pltpu.make_async_copy(w_hbm.at[srcv.at[j]], bufs[b],
                                  sems[b]).wait()
            pltpu.sync_copy(bufs[b], acc_sh.at[dstv.at[j]], add=True)

            @pl.when(jj < NCH // NB - 1)
            def _():
                pltpu.async_copy(w_hbm.at[srcv.at[j + NB]], bufs[b], sems[b])
        return ()

    lax.fori_loop(0, NCH // NB, chunk, ())
    plsc.subcore_barrier()

    @pl.when(s < 10)
    def _():
        pltpu.sync_copy(acc_sh.at[pl.ds(s * 1000, 1000)], obuf)
        pltpu.sync_copy(obuf, out_hbm.at[pl.ds(c * N + s * 1000, 1000)])


def _seg1(srcp, dstp, w):
    k = pl.kernel(
        _seg1_body,
        out_type=jax.ShapeDtypeStruct((NC * N,), _F32),
        mesh=_mesh(),
        scratch_types=[
            pltpu.VMEM((NCH, C), _I32),
            pltpu.VMEM((NCH, C), _I32),
            pltpu.VMEM((C,), _F32),
            pltpu.VMEM((C,), _F32),
            pltpu.VMEM((C,), _F32),
            pltpu.VMEM((C,), _F32),
            pltpu.VMEM((640,), _F32),
            pltpu.VMEM((1000,), _F32),
            pltpu.VMEM_SHARED((ACC1,), _F32),
            pltpu.SemaphoreType.DMA,
            pltpu.SemaphoreType.DMA,
            pltpu.SemaphoreType.DMA,
            pltpu.SemaphoreType.DMA,
        ],
    )
    return k(srcp, dstp, w)


def _segrow_body(src_hbm, dst_hbm, x_hbm, out_hbm, srcv, dstv,
                 v0, v1, v2, v3, zbuf, obuf, acc_sh, s0, s1, s2, s3):
    c = lax.axis_index("c")
    s = lax.axis_index("s")
    w = _wid()
    pltpu.sync_copy(src_hbm.at[w], srcv)
    pltpu.sync_copy(dst_hbm.at[w], dstv)
    z = jnp.zeros((16,), _F32)

    def zrow(i, _):
        for k in range(H // 16):
            zbuf[i, pl.ds(k * 16, 16)] = z
        return ()

    lax.fori_loop(0, C, zrow, ())
    base = s * 632
    for k in range(4):
        pltpu.sync_copy(zbuf, acc_sh.at[pl.ds(base + k * C, C)])

    @pl.when(s < 15)
    def _():
        pltpu.sync_copy(zbuf.at[pl.ds(0, 120)],
                        acc_sh.at[pl.ds(base + 512, 120)])

    @pl.when(s == 15)
    def _():
        pltpu.sync_copy(zbuf.at[pl.ds(0, 24)],
                        acc_sh.at[pl.ds(9480 + 512, 24)])

    plsc.subcore_barrier()

    bufs = (v0, v1, v2, v3)
    sems = (s0, s1, s2, s3)
    for b in range(NB):
        pltpu.async_copy(x_hbm.at[srcv.at[b]], bufs[b], sems[b])

    def chunk(jj, _):
        for b in range(NB):
            j = jj * NB + b
            pltpu.make_async_copy(x_hbm.at[srcv.at[j]], bufs[b],
                                  sems[b]).wait()
            pltpu.sync_copy(bufs[b], acc_sh.at[dstv.at[j]], add=True)

            @pl.when(jj < NCH // NB - 1)
            def _():
                pltpu.async_copy(x_hbm.at[srcv.at[j + NB]], bufs[b], sems[b])
        return ()

    lax.fori_loop(0, NCH // NB, chunk, ())
    plsc.subcore_barrier()

    for k in range(4):
        pltpu.sync_copy(acc_sh.at[pl.ds(base + k * C, C)], obuf)
        pltpu.sync_copy(obuf, out_hbm.at[c, pl.ds(base + k * C, C)])

    @pl.when(s < 15)
    def _():
        pltpu.sync_copy(acc_sh.at[pl.ds(base + 512, 120)],
                        obuf.at[pl.ds(0, 120)])
        pltpu.sync_copy(obuf.at[pl.ds(0, 120)],
                        out_hbm.at[c, pl.ds(base + 512, 120)])

    @pl.when(s == 15)
    def _():
        pltpu.sync_copy(acc_sh.at[pl.ds(9992, 8)], obuf.at[pl.ds(0, 8)])
        pltpu.sync_copy(obuf.at[pl.ds(0, 8)],
                        out_hbm.at[c, pl.ds(9992, 8)])


def _segrow(srcp, dstp, x):
    k = pl.kernel(
        _segrow_body,
        out_type=jax.ShapeDtypeStruct((NC, N, H), _F32),
        mesh=_mesh(),
        compiler_params=pltpu.CompilerParams(use_tc_tiling_on_sc=False),
        scratch_types=[
            pltpu.VMEM((NCH, C), _I32),
            pltpu.VMEM((NCH, C), _I32),
            pltpu.VMEM((C, H), _F32),
            pltpu.VMEM((C, H), _F32),
            pltpu.VMEM((C, H), _F32),
            pltpu.VMEM((C, H), _F32),
            pltpu.VMEM((C, H), _F32),
            pltpu.VMEM((C, H), _F32),
            pltpu.VMEM_SHARED((ACCR, H), _F32),
            pltpu.SemaphoreType.DMA,
            pltpu.SemaphoreType.DMA,
            pltpu.SemaphoreType.DMA,
            pltpu.SemaphoreType.DMA,
        ],
    )
    return k(srcp, dstp, x)


def _dot(a, b):
    return lax.dot_general(a, b, (((1,), (0,)), ((), ())),
                           precision=lax.Precision.HIGHEST,
                           preferred_element_type=_F32)


def _bn_relu(t, g, b):
    m = jnp.mean(t, axis=0, keepdims=True)
    v = jnp.mean((t - m) ** 2, axis=0, keepdims=True)
    return jax.nn.relu((t - m) / jnp.sqrt(v + 1e-5) * g + b)


def _tc1_body(parts, xv, fc1w, fc1b, g1w, g1b, bn1g, bn1b, fc2w, fc2b,
              prob1, xv3):
    aggr1 = parts[0, :] + parts[1, :]
    xv2 = xv[:][:, None] * fc1w[0, :][None, :] + fc1b[0, :][None, :]
    t = aggr1[:, None] * g1w[0, :][None, :] + g1b[0, :][None, :] + xv2
    prob1[...] = _bn_relu(t, bn1g[...], bn1b[...])
    xv3[...] = _dot(xv2, fc2w[...].T) + fc2b[...]


def _tc_mid_body(parts, xvk, gw, gb, bng, bnb, fcw, fcb, probk, xvk1):
    aggr = parts[0] + parts[1]
    t = _dot(aggr, gw[...].T) + gb[...] + xvk[...]
    probk[...] = _bn_relu(t, bng[...], bnb[...])
    xvk1[...] = _dot(xvk[...], fcw[...].T) + fcb[...]


def _tc4_body(parts, xv5, g4w, g4b, q2w, q2b, q3w, q3b, q1w, q1b, q):
    aggr = parts[0] + parts[1]
    prob = jax.nn.relu(_dot(aggr, g4w[...].T) + g4b[...] + xv5[...])
    gf = jnp.mean(prob, axis=0, keepdims=True)
    wgf = _dot(gf, q2w[...].T) + q2b[...]
    wp = _dot(prob, q3w[...].T) + q3b[...]
    qa = q1w[0, :H]
    qb = q1w[0, H:]
    scal = jnp.sum(jax.nn.relu(wgf)[0, :] * qa)
    q[...] = (_dot(jax.nn.relu(wp), qb[:, None]) + scal) + q1b[0, 0]


def _pc(body, out_shapes, *ins):
    return pl.pallas_call(
        body, out_shape=[jax.ShapeDtypeStruct(s, _F32) for s in out_shapes])(*ins)


def kernel(w, edge_index, batch, x_v, params):
    p = params
    src = edge_index[0]
    dst = edge_index[1]
    srcp = jnp.pad(src.reshape(NW, EW), ((0, 0), (0, EPAD - EW)),
                   constant_values=0).reshape(NW, NCH, C)
    dstp = jnp.pad(dst.reshape(NW, EW), ((0, 0), (0, EPAD - EW)),
                   constant_values=N).reshape(NW, NCH, C)

    def r2(a):
        return a.reshape(1, -1)

    parts1 = _seg1(srcp, dstp, w).reshape(NC, N)
    prob1, xv3 = _pc(
        _tc1_body, [(N, H), (N, H)],
        parts1, x_v,
        r2(p['fc1'][0]), r2(p['fc1'][1]), r2(p['g1'][0]), r2(p['g1'][1]),
        r2(p['bn1'][0]), r2(p['bn1'][1]), p['fc2'][0], r2(p['fc2'][1]))

    parts2 = _segrow(srcp, dstp, prob1)
    prob2, xv4 = _pc(
        _tc_mid_body, [(N, H), (N, H)],
        parts2, xv3, p['g2'][0], r2(p['g2'][1]),
        r2(p['bn2'][0]), r2(p['bn2'][1]), p['fc3'][0], r2(p['fc3'][1]))

    parts3 = _segrow(srcp, dstp, prob2)
    prob3, xv5 = _pc(
        _tc_mid_body, [(N, H), (N, H)],
        parts3, xv4, p['g3'][0], r2(p['g3'][1]),
        r2(p['bn3'][0]), r2(p['bn3'][1]), p['fc4'][0], r2(p['fc4'][1]))

    parts4 = _segrow(srcp, dstp, prob3)
    (q,) = _pc(
        _tc4_body, [(N, 1)],
        parts4, xv5, p['g4'][0], r2(p['g4'][1]),
        p['q2'][0], r2(p['q2'][1]), p['q3'][0], r2(p['q3'][1]),
        p['q1'][0].reshape(1, 2 * H), p['q1'][1].reshape(1, 1))

    Q_dense = q[None, :, :]
    Q_mask = jnp.ones((B, N), dtype=bool)
    adj = jnp.zeros((B, N, N), _F32).at[jnp.zeros_like(src), src, dst].add(1.0)
    return (Q_dense, Q_mask, adj)


# speedup vs baseline: 1.0487x; 1.0487x over previous
import functools

import jax
import jax.numpy as jnp
from jax import lax
from jax.experimental import pallas as pl
from jax.experimental.pallas import tpu as pltpu
from jax.experimental.pallas import tpu_sc as plsc

N = 10000
E = 640000
B = 1
H = 64

NC = 2
NS = 16
NW = NC * NS
EW = E // NW
C = 128
NCH = 160
EPAD = NCH * C
ACC1 = NS * 640
ACCR = N + 16

_mesh = functools.partial(
    plsc.VectorSubcoreMesh, core_axis_name="c", subcore_axis_name="s",
    num_cores=NC, num_subcores=NS)

_F32 = jnp.float32
_I32 = jnp.int32


def _wid():
    return lax.axis_index("s") * NC + lax.axis_index("c")


def _seg1_body(src_hbm, dst_hbm, w_hbm, out_hbm, srcv, dstv,
               v0, zbuf, obuf, acc_sh, s0):
    c = lax.axis_index("c")
    s = lax.axis_index("s")
    w = _wid()
    pltpu.sync_copy(src_hbm.at[w], srcv)
    pltpu.sync_copy(dst_hbm.at[w], dstv)
    z = jnp.zeros((16,), _F32)
    for i in range(40):
        zbuf[pl.ds(i * 16, 16)] = z
    pltpu.sync_copy(zbuf, acc_sh.at[pl.ds(s * 640, 640)])
    plsc.subcore_barrier()

    def chunk(j, _):
        pltpu.async_copy(w_hbm.at[srcv.at[j]], v0, s0).wait()
        pltpu.sync_copy(v0, acc_sh.at[dstv.at[j]], add=True)
        return ()

    lax.fori_loop(0, NCH, chunk, ())
    plsc.subcore_barrier()

    @pl.when(s < 10)
    def _():
        pltpu.sync_copy(acc_sh.at[pl.ds(s * 1000, 1000)], obuf)
        pltpu.sync_copy(obuf, out_hbm.at[pl.ds(c * N + s * 1000, 1000)])


def _seg1(srcp, dstp, w):
    k = pl.kernel(
        _seg1_body,
        out_type=jax.ShapeDtypeStruct((NC * N,), _F32),
        mesh=_mesh(),
        scratch_types=[
            pltpu.VMEM((NCH, C), _I32),
            pltpu.VMEM((NCH, C), _I32),
            pltpu.VMEM((C,), _F32),
            pltpu.VMEM((640,), _F32),
            pltpu.VMEM((1000,), _F32),
            pltpu.VMEM_SHARED((ACC1,), _F32),
            pltpu.SemaphoreType.DMA,
        ],
    )
    return k(srcp, dstp, w)


def _segrow_body(src_hbm, dst_hbm, x_hbm, out_hbm, srcv, dstv,
                 v0, zbuf, obuf, acc_sh, s0):
    c = lax.axis_index("c")
    s = lax.axis_index("s")
    w = _wid()
    pltpu.sync_copy(src_hbm.at[w], srcv)
    pltpu.sync_copy(dst_hbm.at[w], dstv)
    z = jnp.zeros((16,), _F32)

    def zrow(i, _):
        for k in range(H // 16):
            zbuf[i, pl.ds(k * 16, 16)] = z
        return ()

    lax.fori_loop(0, C, zrow, ())
    base = s * 632
    for k in range(4):
        pltpu.sync_copy(zbuf, acc_sh.at[pl.ds(base + k * C, C)])

    @pl.when(s < 15)
    def _():
        pltpu.sync_copy(zbuf.at[pl.ds(0, 120)],
                        acc_sh.at[pl.ds(base + 512, 120)])

    @pl.when(s == 15)
    def _():
        pltpu.sync_copy(zbuf.at[pl.ds(0, 24)],
                        acc_sh.at[pl.ds(9480 + 512, 24)])

    plsc.subcore_barrier()

    def chunk(j, _):
        pltpu.async_copy(x_hbm.at[srcv.at[j]], v0, s0).wait()
        pltpu.sync_copy(v0, acc_sh.at[dstv.at[j]], add=True)
        return ()

    lax.fori_loop(0, NCH, chunk, ())
    plsc.subcore_barrier()

    for k in range(4):
        pltpu.sync_copy(acc_sh.at[pl.ds(base + k * C, C)], obuf)
        pltpu.sync_copy(obuf, out_hbm.at[c, pl.ds(base + k * C, C)])

    @pl.when(s < 15)
    def _():
        pltpu.sync_copy(acc_sh.at[pl.ds(base + 512, 120)],
                        obuf.at[pl.ds(0, 120)])
        pltpu.sync_copy(obuf.at[pl.ds(0, 120)],
                        out_hbm.at[c, pl.ds(base + 512, 120)])

    @pl.when(s == 15)
    def _():
        pltpu.sync_copy(acc_sh.at[pl.ds(9992, 8)], obuf.at[pl.ds(0, 8)])
        pltpu.sync_copy(obuf.at[pl.ds(0, 8)],
                        out_hbm.at[c, pl.ds(9992, 8)])


def _segrow(srcp, dstp, x):
    k = pl.kernel(
        _segrow_body,
        out_type=jax.ShapeDtypeStruct((NC, N, H), _F32),
        mesh=_mesh(),
        compiler_params=pltpu.CompilerParams(use_tc_tiling_on_sc=False),
        scratch_types=[
            pltpu.VMEM((NCH, C), _I32),
            pltpu.VMEM((NCH, C), _I32),
            pltpu.VMEM((C, H), _F32),
            pltpu.VMEM((C, H), _F32),
            pltpu.VMEM((C, H), _F32),
            pltpu.VMEM_SHARED((ACCR, H), _F32),
            pltpu.SemaphoreType.DMA,
        ],
    )
    return k(srcp, dstp, x)


RB = 64
RSH = 6
NBK = 157
NBP = 160
CH = RB * N
HSZ = NBP * 16


def _adjsort_body(src_hbm, dst_hbm, sorted_hbm, bs_hbm, srcv, dstv, hist,
                  start, sortv, bstart):
    w = _wid()
    pltpu.sync_copy(src_hbm.at[w], srcv)
    pltpu.sync_copy(dst_hbm.at[w], dstv)
    lane = lax.iota(_I32, 16)
    ones = jnp.ones((16,), _I32)
    zi = jnp.zeros((16,), _I32)
    for i in range(HSZ // 16):
        hist[pl.ds(i * 16, 16)] = zi

    def pass_a(jj, _):
        for u in range(4):
            j = (jj * 4 + u) * 16
            b = srcv[pl.ds(j, 16)] >> RSH
            plsc.addupdate_scatter(hist, [b * 16 + lane], ones)
        return ()

    lax.fori_loop(0, EPAD // 64, pass_a, ())

    def prefix(b, carry):
        v = hist[pl.ds(b * 16, 16)]
        cs = plsc.cumsum(v)
        start[pl.ds(b * 16, 16)] = cs - v + carry
        return carry + jnp.sum(v)

    lax.fori_loop(0, NBP, prefix, jnp.int32(0))
    for k in range(NBP // 16):
        bb = (lane + 16 * k) * 16
        bstart[pl.ds(16 * k, 16)] = plsc.load_gather(start, [bb])

    def pass_b(jj, _):
        for u in range(4):
            j = (jj * 4 + u) * 16
            sv = srcv[pl.ds(j, 16)]
            dv = dstv[pl.ds(j, 16)]
            b = sv >> RSH
            idx = b * 16 + lane
            loc = (sv & (RB - 1)) * N + dv
            pos = plsc.load_gather(start, [idx])
            plsc.store_scatter(sortv, [pos], loc)
            plsc.addupdate_scatter(start, [idx], ones)
        return ()

    lax.fori_loop(0, EPAD // 64, pass_b, ())
    pltpu.sync_copy(sortv, sorted_hbm.at[pl.ds(w * EPAD, EPAD)])
    pltpu.sync_copy(bstart, bs_hbm.at[pl.ds(w * NBP, NBP)])


def _adjsort(srcA, dstA):
    k = pl.kernel(
        _adjsort_body,
        out_type=(jax.ShapeDtypeStruct((NW * EPAD + 128,), _I32),
                  jax.ShapeDtypeStruct((NW * NBP,), _I32)),
        mesh=_mesh(),
        compiler_params=pltpu.CompilerParams(use_tc_tiling_on_sc=False,
                                             needs_layout_passes=False),
        scratch_types=[
            pltpu.VMEM((EPAD,), _I32),
            pltpu.VMEM((EPAD,), _I32),
            pltpu.VMEM((HSZ,), _I32),
            pltpu.VMEM((HSZ,), _I32),
            pltpu.VMEM((EPAD,), _I32),
            pltpu.VMEM((NBP,), _I32),
        ],
    )
    return k(srcA, dstA)


def _adjscat_body(sorted_hbm, bs_hbm, adj_hbm, bsv, zbuf, obuf, locv, idxv,
                  valv, chunk_sh):
    c = lax.axis_index("c")
    s = lax.axis_index("s")
    lane = lax.iota(_I32, 16)
    z = jnp.zeros((16,), _F32)

    def zrow(i, _):
        zbuf[pl.ds(i * 16, 16)] = z
        return ()

    lax.fori_loop(0, 2500, zrow, ())
    for ti in range(2):
        pltpu.sync_copy(bs_hbm.at[pl.ds((s * 2 + ti) * NBP, NBP)],
                        bsv.at[ti, pl.ds(0, NBP)])

    def do_chunk(i, _):
        cc = 2 * i + c
        pltpu.sync_copy(zbuf, chunk_sh.at[pl.ds(s * 40000, 40000)])
        plsc.subcore_barrier()
        for ti in range(2):
            t = s * 2 + ti
            vv = bsv[ti, pl.ds(cc, 16)]
            st = vv[0]
            en = vv[1]
            st0 = jnp.bitwise_and(st, -8)
            g0 = pl.multiple_of(t * EPAD + st0, 8)
            nch = (en - st0 + 127) >> 7

            def inner(k, _):
                pltpu.sync_copy(
                    sorted_hbm.at[pl.ds(pl.multiple_of(g0 + k * 128, 8),
                                        128)], locv)
                p0 = st0 + k * 128
                for m in range(8):
                    lv = locv[pl.ds(16 * m, 16)]
                    pv = p0 + 16 * m + lane
                    valid = (pv >= st) & (pv < en)
                    idxv[pl.ds(16 * m, 16)] = jnp.minimum(
                        jnp.maximum(lv, 0), CH - 1)
                    valv[pl.ds(16 * m, 16)] = jnp.where(valid, 1.0, 0.0)
                pltpu.sync_copy(valv, chunk_sh.at[idxv], add=True)
                return ()

            lax.fori_loop(0, nch, inner, ())
        plsc.subcore_barrier()

        @pl.when(cc < NBK - 1)
        def _():
            off = s * 40000
            pltpu.sync_copy(chunk_sh.at[pl.ds(off, 40000)], obuf)
            pltpu.sync_copy(
                obuf,
                adj_hbm.at[pl.ds(pl.multiple_of(cc * CH + off, 8), 40000)])

        @pl.when(cc == NBK - 1)
        def _():
            pltpu.sync_copy(chunk_sh.at[pl.ds(s * 10000, 10000)],
                            obuf.at[pl.ds(0, 10000)])
            pltpu.sync_copy(obuf.at[pl.ds(0, 10000)],
                            adj_hbm.at[pl.ds(
                                pl.multiple_of(cc * CH + s * 10000, 8),
                                10000)])

        plsc.subcore_barrier()
        return ()

    lax.fori_loop(0, 79 - c, do_chunk, ())


def _adjscat(sorted_e, bs):
    k = pl.kernel(
        _adjscat_body,
        out_type=jax.ShapeDtypeStruct((N * N,), _F32),
        mesh=_mesh(),
        compiler_params=pltpu.CompilerParams(use_tc_tiling_on_sc=False,
                                             needs_layout_passes=False),
        scratch_types=[
            pltpu.VMEM((2, NBP + 16), _I32),
            pltpu.VMEM((40000,), _F32),
            pltpu.VMEM((40000,), _F32),
            pltpu.VMEM((128,), _I32),
            pltpu.VMEM((128,), _I32),
            pltpu.VMEM((128,), _F32),
            pltpu.VMEM_SHARED((CH,), _F32),
        ],
    )
    return k(sorted_e, bs)


def _dot(a, b):
    return lax.dot_general(a, b, (((1,), (0,)), ((), ())),
                           precision=lax.Precision.HIGHEST,
                           preferred_element_type=_F32)


def _bn_relu(t, g, b):
    m = jnp.mean(t, axis=0, keepdims=True)
    v = jnp.mean((t - m) ** 2, axis=0, keepdims=True)
    return jax.nn.relu((t - m) / jnp.sqrt(v + 1e-5) * g + b)


def _tc1_body(parts, xv, fc1w, fc1b, g1w, g1b, bn1g, bn1b, fc2w, fc2b,
              prob1, xv3):
    aggr1 = parts[0, :] + parts[1, :]
    xv2 = xv[:][:, None] * fc1w[0, :][None, :] + fc1b[0, :][None, :]
    t = aggr1[:, None] * g1w[0, :][None, :] + g1b[0, :][None, :] + xv2
    prob1[...] = _bn_relu(t, bn1g[...], bn1b[...])
    xv3[...] = _dot(xv2, fc2w[...].T) + fc2b[...]


def _tc_mid_body(parts, xvk, gw, gb, bng, bnb, fcw, fcb, probk, xvk1):
    aggr = parts[0] + parts[1]
    t = _dot(aggr, gw[...].T) + gb[...] + xvk[...]
    probk[...] = _bn_relu(t, bng[...], bnb[...])
    xvk1[...] = _dot(xvk[...], fcw[...].T) + fcb[...]


def _tc4_body(parts, xv5, g4w, g4b, q2w, q2b, q3w, q3b, q1w, q1b, q):
    aggr = parts[0] + parts[1]
    prob = jax.nn.relu(_dot(aggr, g4w[...].T) + g4b[...] + xv5[...])
    gf = jnp.mean(prob, axis=0, keepdims=True)
    wgf = _dot(gf, q2w[...].T) + q2b[...]
    wp = _dot(prob, q3w[...].T) + q3b[...]
    qa = q1w[0, :H]
    qb = q1w[0, H:]
    scal = jnp.sum(jax.nn.relu(wgf)[0, :] * qa)
    q[...] = (_dot(jax.nn.relu(wp), qb[:, None]) + scal) + q1b[0, 0]


def _pc(body, out_shapes, *ins):
    return pl.pallas_call(
        body, out_shape=[jax.ShapeDtypeStruct(s, _F32) for s in out_shapes])(*ins)


def kernel(w, edge_index, batch, x_v, params):
    p = params
    src = edge_index[0]
    dst = edge_index[1]
    srcp = jnp.pad(src.reshape(NW, EW), ((0, 0), (0, EPAD - EW)),
                   constant_values=0).reshape(NW, NCH, C)
    dstp = jnp.pad(dst.reshape(NW, EW), ((0, 0), (0, EPAD - EW)),
                   constant_values=N).reshape(NW, NCH, C)

    def r2(a):
        return a.reshape(1, -1)

    parts1 = _seg1(srcp, dstp, w).reshape(NC, N)
    prob1, xv3 = _pc(
        _tc1_body, [(N, H), (N, H)],
        parts1, x_v,
        r2(p['fc1'][0]), r2(p['fc1'][1]), r2(p['g1'][0]), r2(p['g1'][1]),
        r2(p['bn1'][0]), r2(p['bn1'][1]), p['fc2'][0], r2(p['fc2'][1]))

    parts2 = _segrow(srcp, dstp, prob1)
    prob2, xv4 = _pc(
        _tc_mid_body, [(N, H), (N, H)],
        parts2, xv3, p['g2'][0], r2(p['g2'][1]),
        r2(p['bn2'][0]), r2(p['bn2'][1]), p['fc3'][0], r2(p['fc3'][1]))

    parts3 = _segrow(srcp, dstp, prob2)
    prob3, xv5 = _pc(
        _tc_mid_body, [(N, H), (N, H)],
        parts3, xv4, p['g3'][0], r2(p['g3'][1]),
        r2(p['bn3'][0]), r2(p['bn3'][1]), p['fc4'][0], r2(p['fc4'][1]))

    parts4 = _segrow(srcp, dstp, prob3)
    (q,) = _pc(
        _tc4_body, [(N, 1)],
        parts4, xv5, p['g4'][0], r2(p['g4'][1]),
        p['q2'][0], r2(p['q2'][1]), p['q3'][0], r2(p['q3'][1]),
        p['q1'][0].reshape(1, 2 * H), p['q1'][1].reshape(1, 1))

    srcA = jnp.pad(src.reshape(NW, EW), ((0, 0), (0, EPAD - EW)),
                   constant_values=N)
    dstA = jnp.pad(dst.reshape(NW, EW), ((0, 0), (0, EPAD - EW)),
                   constant_values=0)
    sorted_e, bs = _adjsort(srcA, dstA)
    adj = _adjscat(sorted_e, bs).reshape(1, N, N)

    Q_dense = q[None, :, :]
    Q_mask = jnp.ones((B, N), dtype=bool)
    return (Q_dense, Q_mask, adj)


# speedup vs baseline: 1.1667x; 1.1125x over previous
import functools

import jax
import jax.numpy as jnp
from jax import lax
from jax.experimental import pallas as pl
from jax.experimental.pallas import tpu as pltpu
from jax.experimental.pallas import tpu_sc as plsc

N = 10000
E = 640000
B = 1
H = 64

NC = 2
NS = 16
NW = NC * NS
EW = E // NW
C = 128
NCH = 160
EPAD = NCH * C
ACC1 = NS * 640
ACCR = N + 16

_mesh = functools.partial(
    plsc.VectorSubcoreMesh, core_axis_name="c", subcore_axis_name="s",
    num_cores=NC, num_subcores=NS)

_F32 = jnp.float32
_I32 = jnp.int32


def _wid():
    return lax.axis_index("s") * NC + lax.axis_index("c")


def _seg1_body(src_hbm, dst_hbm, w_hbm, out_hbm, srcv, dstv,
               v0, v1, v2, v3, zbuf, obuf, acc_sh,
               g0, g1, g2, g3, t0, t1, t2, t3):
    c = lax.axis_index("c")
    s = lax.axis_index("s")
    w = _wid()
    pltpu.sync_copy(src_hbm.at[w], srcv)
    pltpu.sync_copy(dst_hbm.at[w], dstv)
    z = jnp.zeros((16,), _F32)
    for i in range(40):
        zbuf[pl.ds(i * 16, 16)] = z
    pltpu.sync_copy(zbuf, acc_sh.at[pl.ds(s * 640, 640)])
    plsc.subcore_barrier()

    bufs = (v0, v1, v2, v3)
    gs = (g0, g1, g2, g3)
    ss = (t0, t1, t2, t3)
    pltpu.async_copy(w_hbm.at[srcv.at[0]], bufs[0], gs[0])
    pltpu.async_copy(w_hbm.at[srcv.at[1]], bufs[1], gs[1])

    def chunk(jj, _):
        for b in range(4):
            j = jj * 4 + b
            bn = (b + 2) % 4
            pltpu.make_async_copy(w_hbm.at[srcv.at[j]], bufs[b],
                                  gs[b]).wait()
            pltpu.async_copy(bufs[b], acc_sh.at[dstv.at[j]], ss[b],
                             add=True)

            @pl.when(j >= 2)
            def _():
                pltpu.make_async_copy(bufs[bn], acc_sh.at[dstv.at[j - 2]],
                                      ss[bn]).wait()

            @pl.when(j + 2 < NCH)
            def _():
                pltpu.async_copy(w_hbm.at[srcv.at[j + 2]], bufs[bn], gs[bn])
        return ()

    lax.fori_loop(0, NCH // 4, chunk, ())
    pltpu.make_async_copy(bufs[2], acc_sh.at[dstv.at[NCH - 2]],
                          ss[2]).wait()
    pltpu.make_async_copy(bufs[3], acc_sh.at[dstv.at[NCH - 1]],
                          ss[3]).wait()
    plsc.subcore_barrier()

    @pl.when(s < 10)
    def _():
        pltpu.sync_copy(acc_sh.at[pl.ds(s * 1000, 1000)], obuf)
        pltpu.sync_copy(obuf, out_hbm.at[pl.ds(c * N + s * 1000, 1000)])


def _seg1(srcp, dstp, w):
    k = pl.kernel(
        _seg1_body,
        out_type=jax.ShapeDtypeStruct((NC * N,), _F32),
        mesh=_mesh(),
        scratch_types=(
            [pltpu.VMEM((NCH, C), _I32), pltpu.VMEM((NCH, C), _I32)]
            + [pltpu.VMEM((C,), _F32)] * 4
            + [pltpu.VMEM((640,), _F32), pltpu.VMEM((1000,), _F32)]
            + [pltpu.VMEM_SHARED((ACC1,), _F32)]
            + [pltpu.SemaphoreType.DMA] * 8
        ),
    )
    return k(srcp, dstp, w)


def _segrow_body(src_hbm, dst_hbm, x_hbm, out_hbm, srcv, dstv,
                 v0, v1, v2, v3, zbuf, obuf, acc_sh,
                 g0, g1, g2, g3, t0, t1, t2, t3):
    c = lax.axis_index("c")
    s = lax.axis_index("s")
    w = _wid()
    pltpu.sync_copy(src_hbm.at[w], srcv)
    pltpu.sync_copy(dst_hbm.at[w], dstv)
    z = jnp.zeros((16,), _F32)

    def zrow(i, _):
        for k in range(H // 16):
            zbuf[i, pl.ds(k * 16, 16)] = z
        return ()

    lax.fori_loop(0, C, zrow, ())
    base = s * 632
    for k in range(4):
        pltpu.sync_copy(zbuf, acc_sh.at[pl.ds(base + k * C, C)])

    @pl.when(s < 15)
    def _():
        pltpu.sync_copy(zbuf.at[pl.ds(0, 120)],
                        acc_sh.at[pl.ds(base + 512, 120)])

    @pl.when(s == 15)
    def _():
        pltpu.sync_copy(zbuf.at[pl.ds(0, 24)],
                        acc_sh.at[pl.ds(9480 + 512, 24)])

    plsc.subcore_barrier()

    bufs = (v0, v1, v2, v3)
    gs = (g0, g1, g2, g3)
    ss = (t0, t1, t2, t3)
    pltpu.async_copy(x_hbm.at[srcv.at[0]], bufs[0], gs[0])
    pltpu.async_copy(x_hbm.at[srcv.at[1]], bufs[1], gs[1])

    def chunk(jj, _):
        for b in range(4):
            j = jj * 4 + b
            bn = (b + 2) % 4
            pltpu.make_async_copy(x_hbm.at[srcv.at[j]], bufs[b],
                                  gs[b]).wait()
            pltpu.async_copy(bufs[b], acc_sh.at[dstv.at[j]], ss[b],
                             add=True)

            @pl.when(j >= 2)
            def _():
                pltpu.make_async_copy(bufs[bn], acc_sh.at[dstv.at[j - 2]],
                                      ss[bn]).wait()

            @pl.when(j + 2 < NCH)
            def _():
                pltpu.async_copy(x_hbm.at[srcv.at[j + 2]], bufs[bn], gs[bn])
        return ()

    lax.fori_loop(0, NCH // 4, chunk, ())
    pltpu.make_async_copy(bufs[2], acc_sh.at[dstv.at[NCH - 2]],
                          ss[2]).wait()
    pltpu.make_async_copy(bufs[3], acc_sh.at[dstv.at[NCH - 1]],
                          ss[3]).wait()
    plsc.subcore_barrier()

    for k in range(4):
        pltpu.sync_copy(acc_sh.at[pl.ds(base + k * C, C)], obuf)
        pltpu.sync_copy(obuf, out_hbm.at[c, pl.ds(base + k * C, C)])

    @pl.when(s < 15)
    def _():
        pltpu.sync_copy(acc_sh.at[pl.ds(base + 512, 120)],
                        obuf.at[pl.ds(0, 120)])
        pltpu.sync_copy(obuf.at[pl.ds(0, 120)],
                        out_hbm.at[c, pl.ds(base + 512, 120)])

    @pl.when(s == 15)
    def _():
        pltpu.sync_copy(acc_sh.at[pl.ds(9992, 8)], obuf.at[pl.ds(0, 8)])
        pltpu.sync_copy(obuf.at[pl.ds(0, 8)],
                        out_hbm.at[c, pl.ds(9992, 8)])


def _segrow(srcp, dstp, x):
    k = pl.kernel(
        _segrow_body,
        out_type=jax.ShapeDtypeStruct((NC, N, H), _F32),
        mesh=_mesh(),
        compiler_params=pltpu.CompilerParams(use_tc_tiling_on_sc=False),
        scratch_types=(
            [pltpu.VMEM((NCH, C), _I32), pltpu.VMEM((NCH, C), _I32)]
            + [pltpu.VMEM((C, H), _F32)] * 6
            + [pltpu.VMEM_SHARED((ACCR, H), _F32)]
            + [pltpu.SemaphoreType.DMA] * 8
        ),
    )
    return k(srcp, dstp, x)


RB = 64
RSH = 6
NBK = 157
NBP = 160
CH = RB * N
HSZ = NBP * 16


def _adjsort_body(src_hbm, dst_hbm, sorted_hbm, bs_hbm, srcv, dstv, hist,
                  start, sortv, bstart):
    w = _wid()
    pltpu.sync_copy(src_hbm.at[w], srcv)
    pltpu.sync_copy(dst_hbm.at[w], dstv)
    lane = lax.iota(_I32, 16)
    ones = jnp.ones((16,), _I32)
    zi = jnp.zeros((16,), _I32)
    for i in range(HSZ // 16):
        hist[pl.ds(i * 16, 16)] = zi

    def pass_a(jj, _):
        for u in range(4):
            j = (jj * 4 + u) * 16
            b = srcv[pl.ds(j, 16)] >> RSH
            plsc.addupdate_scatter(hist, [b * 16 + lane], ones)
        return ()

    lax.fori_loop(0, EPAD // 64, pass_a, ())

    def prefix(b, carry):
        v = hist[pl.ds(b * 16, 16)]
        cs = plsc.cumsum(v)
        start[pl.ds(b * 16, 16)] = cs - v + carry
        return carry + jnp.sum(v)

    lax.fori_loop(0, NBP, prefix, jnp.int32(0))
    for k in range(NBP // 16):
        bb = (lane + 16 * k) * 16
        bstart[pl.ds(16 * k, 16)] = plsc.load_gather(start, [bb])

    def pass_b(jj, _):
        for u in range(4):
            j = (jj * 4 + u) * 16
            sv = srcv[pl.ds(j, 16)]
            dv = dstv[pl.ds(j, 16)]
            b = sv >> RSH
            idx = b * 16 + lane
            loc = (sv & (RB - 1)) * N + dv
            pos = plsc.load_gather(start, [idx])
            plsc.store_scatter(sortv, [pos], loc)
            plsc.addupdate_scatter(start, [idx], ones)
        return ()

    lax.fori_loop(0, EPAD // 64, pass_b, ())
    pltpu.sync_copy(sortv, sorted_hbm.at[pl.ds(w * EPAD, EPAD)])
    pltpu.sync_copy(bstart, bs_hbm.at[pl.ds(w * NBP, NBP)])


def _adjsort(srcA, dstA):
    k = pl.kernel(
        _adjsort_body,
        out_type=(jax.ShapeDtypeStruct((NW * EPAD + 128,), _I32),
                  jax.ShapeDtypeStruct((NW * NBP,), _I32)),
        mesh=_mesh(),
        compiler_params=pltpu.CompilerParams(use_tc_tiling_on_sc=False,
                                             needs_layout_passes=False),
        scratch_types=[
            pltpu.VMEM((EPAD,), _I32),
            pltpu.VMEM((EPAD,), _I32),
            pltpu.VMEM((HSZ,), _I32),
            pltpu.VMEM((HSZ,), _I32),
            pltpu.VMEM((EPAD,), _I32),
            pltpu.VMEM((NBP,), _I32),
        ],
    )
    return k(srcA, dstA)


def _adjscat_body(sorted_hbm, bs_hbm, adj_hbm, bsv, zbuf, obuf, locv, idxv,
                  valv, chunk_sh):
    c = lax.axis_index("c")
    s = lax.axis_index("s")
    lane = lax.iota(_I32, 16)
    z = jnp.zeros((16,), _F32)

    def zrow(i, _):
        zbuf[pl.ds(i * 16, 16)] = z
        return ()

    lax.fori_loop(0, 2500, zrow, ())
    for ti in range(2):
        pltpu.sync_copy(bs_hbm.at[pl.ds((s * 2 + ti) * NBP, NBP)],
                        bsv.at[ti, pl.ds(0, NBP)])

    def do_chunk(i, _):
        cc = 2 * i + c
        pltpu.sync_copy(zbuf, chunk_sh.at[pl.ds(s * 40000, 40000)])
        plsc.subcore_barrier()
        for ti in range(2):
            t = s * 2 + ti
            vv = bsv[ti, pl.ds(cc, 16)]
            st = vv[0]
            en = vv[1]
            st0 = jnp.bitwise_and(st, -8)
            g0 = pl.multiple_of(t * EPAD + st0, 8)
            nch = (en - st0 + 127) >> 7

            def inner(k, _):
                pltpu.sync_copy(
                    sorted_hbm.at[pl.ds(pl.multiple_of(g0 + k * 128, 8),
                                        128)], locv)
                p0 = st0 + k * 128
                for m in range(8):
                    lv = locv[pl.ds(16 * m, 16)]
                    pv = p0 + 16 * m + lane
                    valid = (pv >= st) & (pv < en)
                    idxv[pl.ds(16 * m, 16)] = jnp.minimum(
                        jnp.maximum(lv, 0), CH - 1)
                    valv[pl.ds(16 * m, 16)] = jnp.where(valid, 1.0, 0.0)
                pltpu.sync_copy(valv, chunk_sh.at[idxv], add=True)
                return ()

            lax.fori_loop(0, nch, inner, ())
        plsc.subcore_barrier()

        @pl.when(cc < NBK - 1)
        def _():
            off = s * 40000
            pltpu.sync_copy(chunk_sh.at[pl.ds(off, 40000)], obuf)
            pltpu.sync_copy(
                obuf,
                adj_hbm.at[pl.ds(pl.multiple_of(cc * CH + off, 8), 40000)])

        @pl.when(cc == NBK - 1)
        def _():
            pltpu.sync_copy(chunk_sh.at[pl.ds(s * 10000, 10000)],
                            obuf.at[pl.ds(0, 10000)])
            pltpu.sync_copy(obuf.at[pl.ds(0, 10000)],
                            adj_hbm.at[pl.ds(
                                pl.multiple_of(cc * CH + s * 10000, 8),
                                10000)])

        plsc.subcore_barrier()
        return ()

    lax.fori_loop(0, 79 - c, do_chunk, ())


def _adjscat(sorted_e, bs):
    k = pl.kernel(
        _adjscat_body,
        out_type=jax.ShapeDtypeStruct((N * N,), _F32),
        mesh=_mesh(),
        compiler_params=pltpu.CompilerParams(use_tc_tiling_on_sc=False,
                                             needs_layout_passes=False),
        scratch_types=[
            pltpu.VMEM((2, NBP + 16), _I32),
            pltpu.VMEM((40000,), _F32),
            pltpu.VMEM((40000,), _F32),
            pltpu.VMEM((128,), _I32),
            pltpu.VMEM((128,), _I32),
            pltpu.VMEM((128,), _F32),
            pltpu.VMEM_SHARED((CH,), _F32),
        ],
    )
    return k(sorted_e, bs)


def _dot(a, b):
    return lax.dot_general(a, b, (((1,), (0,)), ((), ())),
                           precision=lax.Precision.HIGHEST,
                           preferred_element_type=_F32)


def _bn_relu(t, g, b):
    m = jnp.mean(t, axis=0, keepdims=True)
    v = jnp.mean((t - m) ** 2, axis=0, keepdims=True)
    return jax.nn.relu((t - m) / jnp.sqrt(v + 1e-5) * g + b)


def _tc1_body(parts, xv, fc1w, fc1b, g1w, g1b, bn1g, bn1b, fc2w, fc2b,
              prob1, xv3):
    aggr1 = parts[0, :] + parts[1, :]
    xv2 = xv[:][:, None] * fc1w[0, :][None, :] + fc1b[0, :][None, :]
    t = aggr1[:, None] * g1w[0, :][None, :] + g1b[0, :][None, :] + xv2
    prob1[...] = _bn_relu(t, bn1g[...], bn1b[...])
    xv3[...] = _dot(xv2, fc2w[...].T) + fc2b[...]


def _tc_mid_body(parts, xvk, gw, gb, bng, bnb, fcw, fcb, probk, xvk1):
    aggr = parts[0] + parts[1]
    t = _dot(aggr, gw[...].T) + gb[...] + xvk[...]
    probk[...] = _bn_relu(t, bng[...], bnb[...])
    xvk1[...] = _dot(xvk[...], fcw[...].T) + fcb[...]


def _tc4_body(parts, xv5, g4w, g4b, q2w, q2b, q3w, q3b, q1w, q1b, q):
    aggr = parts[0] + parts[1]
    prob = jax.nn.relu(_dot(aggr, g4w[...].T) + g4b[...] + xv5[...])
    gf = jnp.mean(prob, axis=0, keepdims=True)
    wgf = _dot(gf, q2w[...].T) + q2b[...]
    wp = _dot(prob, q3w[...].T) + q3b[...]
    qa = q1w[0, :H]
    qb = q1w[0, H:]
    scal = jnp.sum(jax.nn.relu(wgf)[0, :] * qa)
    q[...] = (_dot(jax.nn.relu(wp), qb[:, None]) + scal) + q1b[0, 0]


def _pc(body, out_shapes, *ins):
    return pl.pallas_call(
        body, out_shape=[jax.ShapeDtypeStruct(s, _F32) for s in out_shapes])(*ins)


def kernel(w, edge_index, batch, x_v, params):
    p = params
    src = edge_index[0]
    dst = edge_index[1]
    srcp = jnp.pad(src.reshape(NW, EW), ((0, 0), (0, EPAD - EW)),
                   constant_values=0).reshape(NW, NCH, C)
    dstp = jnp.pad(dst.reshape(NW, EW), ((0, 0), (0, EPAD - EW)),
                   constant_values=N).reshape(NW, NCH, C)

    def r2(a):
        return a.reshape(1, -1)

    parts1 = _seg1(srcp, dstp, w).reshape(NC, N)
    prob1, xv3 = _pc(
        _tc1_body, [(N, H), (N, H)],
        parts1, x_v,
        r2(p['fc1'][0]), r2(p['fc1'][1]), r2(p['g1'][0]), r2(p['g1'][1]),
        r2(p['bn1'][0]), r2(p['bn1'][1]), p['fc2'][0], r2(p['fc2'][1]))

    parts2 = _segrow(srcp, dstp, prob1)
    prob2, xv4 = _pc(
        _tc_mid_body, [(N, H), (N, H)],
        parts2, xv3, p['g2'][0], r2(p['g2'][1]),
        r2(p['bn2'][0]), r2(p['bn2'][1]), p['fc3'][0], r2(p['fc3'][1]))

    parts3 = _segrow(srcp, dstp, prob2)
    prob3, xv5 = _pc(
        _tc_mid_body, [(N, H), (N, H)],
        parts3, xv4, p['g3'][0], r2(p['g3'][1]),
        r2(p['bn3'][0]), r2(p['bn3'][1]), p['fc4'][0], r2(p['fc4'][1]))

    parts4 = _segrow(srcp, dstp, prob3)
    (q,) = _pc(
        _tc4_body, [(N, 1)],
        parts4, xv5, p['g4'][0], r2(p['g4'][1]),
        p['q2'][0], r2(p['q2'][1]), p['q3'][0], r2(p['q3'][1]),
        p['q1'][0].reshape(1, 2 * H), p['q1'][1].reshape(1, 1))

    srcA = jnp.pad(src.reshape(NW, EW), ((0, 0), (0, EPAD - EW)),
                   constant_values=N)
    dstA = jnp.pad(dst.reshape(NW, EW), ((0, 0), (0, EPAD - EW)),
                   constant_values=0)
    sorted_e, bs = _adjsort(srcA, dstA)
    adj = _adjscat(sorted_e, bs).reshape(1, N, N)

    Q_dense = q[None, :, :]
    Q_mask = jnp.ones((B, N), dtype=bool)
    return (Q_dense, Q_mask, adj)
